# baseline (device time: 54057 ns/iter reference)
import jax
import jax.numpy as jnp
from jax import lax
from jax.experimental import pallas as pl
from jax.experimental.pallas import tpu as pltpu

T = 2048
D = 1024
Y = 4
Z = 4
NB = Y * Z
BLK = T // NB
HALF = BLK // 2
MESH = pl.DeviceIdType.MESH


def kernel(ids, E, stage=3):
    V = E.shape[0]

    def body(ids_smem, ids_vmem, e_hbm, out_ref,
             own_ref, sendx_ref, recvx_ref, a_ref, b_ref,
             gather_sem, sendx_sem, recvx_sem, p1_sems, p2_sems,
             xf_send, xf_recv):
        my_x = lax.axis_index("x")
        my_y = lax.axis_index("y")
        my_z = lax.axis_index("z")
        partner = (1 - my_x, my_y, my_z)
        offset = my_x * V
        b = my_y * Z + my_z
        base = b * BLK
        fwd_out = 3 * my_x
        fwd_in = 3 - 3 * my_x

        barrier = pltpu.get_barrier_semaphore()
        pl.semaphore_signal(barrier, inc=1, device_id=partner,
                            device_id_type=MESH)
        n_nbr = jnp.int32(1)
        for k in range(1, max(Y, Z)):
            for cond, tgt in [
                (my_z - k >= 0, (my_x, my_y, my_z - k)),
                (my_z + k <= Z - 1, (my_x, my_y, my_z + k)),
                (my_y - k >= 0, (my_x, my_y - k, my_z)),
                (my_y + k <= Y - 1, (my_x, my_y + k, my_z)),
            ]:
                @pl.when(cond)
                def _(tgt=tgt):
                    pl.semaphore_signal(barrier, inc=1, device_id=tgt,
                                        device_id_type=MESH)
                n_nbr = n_nbr + cond.astype(jnp.int32)

        UNROLL = 4

        def issue(i, carry):
            for j in range(UNROLL):
                t = i * UNROLL + j
                local = ids_smem[base + t] - offset
                c = jnp.clip(local, 0, V - 1)
                pltpu.make_async_copy(
                    e_hbm.at[pl.ds(c, 1), :],
                    own_ref.at[pl.ds(t, 1), :],
                    gather_sem,
                ).start()
            return carry

        lax.fori_loop(0, BLK // UNROLL, issue, 0)
        pltpu.make_async_copy(
            e_hbm.at[pl.ds(0, BLK), :], own_ref.at[:, :], gather_sem
        ).wait()

        ids_b = ids_vmem[pl.ds(base, BLK), :]
        in_range = (ids_b >= offset) & (ids_b < offset + V)
        masked = jnp.where(in_range, own_ref[:, :], 0.0)
        sendx_ref[:, :] = masked.astype(jnp.bfloat16)

        pl.semaphore_wait(barrier, n_nbr)

        xr = pltpu.make_async_remote_copy(
            src_ref=sendx_ref, dst_ref=recvx_ref,
            send_sem=sendx_sem, recv_sem=recvx_sem,
            device_id=partner, device_id_type=MESH,
        )
        xr.start()
        xr.wait()
        summed = masked + recvx_ref[:, :].astype(jnp.float32)
        summed_bf = summed.astype(jnp.bfloat16)
        a_ref[pl.ds(my_y, 1), pl.ds(my_z, 1), :, :] = (
            summed_bf[:HALF][None, None]
        )
        b_ref[pl.ds(my_z, 1), pl.ds(my_y, 1), :, :] = (
            summed_bf[HALF:][None, None]
        )

        z_nbr = lambda d: (my_x, my_y, my_z + d)
        y_nbr = lambda d: (my_x, my_y + d, my_z)
        a1_chunk = lambda p: a_ref.at[my_y, p]
        b1_chunk = lambda p: b_ref.at[my_z, p]
        a2_chunk = lambda p: a_ref.at[p, pl.ds(my_x, 3)]
        b2_chunk = lambda p: b_ref.at[p, pl.ds(my_x, 3)]
        a2_fout = lambda p: a_ref.at[p, fwd_out]
        b2_fout = lambda p: b_ref.at[p, fwd_out]
        a2_fin = lambda p: a_ref.at[p, fwd_in]
        b2_fin = lambda p: b_ref.at[p, fwd_in]
        sems1 = [tuple(p1_sems.at[i] for i in range(4 * f, 4 * f + 4))
                 for f in range(2)]
        sems2 = [tuple(p2_sems.at[i] for i in range(4 * f, 4 * f + 4))
                 for f in range(2)]
        flows1 = [(my_z, a1_chunk, z_nbr, sems1[0]),
                  (my_y, b1_chunk, y_nbr, sems1[1])]
        flows2 = [(my_y, a2_chunk, a2_fout, a2_fin, y_nbr, sems2[0]),
                  (my_z, b2_chunk, b2_fout, b2_fin, z_nbr, sems2[1])]

        def unicast_sends(flows, n):
            for pos, chunk_at, nbr, (r_s, r_r, l_s, l_r) in flows:
                for k in range(1, n):
                    @pl.when(pos + k <= n - 1)
                    def _(k=k, pos=pos, chunk_at=chunk_at, nbr=nbr,
                          r_s=r_s, r_r=r_r):
                        pltpu.make_async_remote_copy(
                            src_ref=chunk_at(pos), dst_ref=chunk_at(pos),
                            send_sem=r_s.at[k - 1], recv_sem=r_r.at[k - 1],
                            device_id=nbr(k), device_id_type=MESH,
                        ).start()

                    @pl.when(pos - k >= 0)
                    def _(k=k, pos=pos, chunk_at=chunk_at, nbr=nbr,
                          l_s=l_s, l_r=l_r):
                        pltpu.make_async_remote_copy(
                            src_ref=chunk_at(pos), dst_ref=chunk_at(pos),
                            send_sem=l_s.at[k - 1], recv_sem=l_r.at[k - 1],
                            device_id=nbr(-k), device_id_type=MESH,
                        ).start()

        def unicast_send_drains(flows, n):
            for pos, chunk_at, nbr, (r_s, r_r, l_s, l_r) in flows:
                for k in range(1, n):
                    @pl.when(pos + k <= n - 1)
                    def _(k=k, pos=pos, chunk_at=chunk_at, nbr=nbr,
                          r_s=r_s, r_r=r_r):
                        pltpu.make_async_remote_copy(
                            src_ref=chunk_at(pos), dst_ref=chunk_at(pos),
                            send_sem=r_s.at[k - 1], recv_sem=r_r.at[k - 1],
                            device_id=nbr(k), device_id_type=MESH,
                        ).wait_send()

                    @pl.when(pos - k >= 0)
                    def _(k=k, pos=pos, chunk_at=chunk_at, nbr=nbr,
                          l_s=l_s, l_r=l_r):
                        pltpu.make_async_remote_copy(
                            src_ref=chunk_at(pos), dst_ref=chunk_at(pos),
                            send_sem=l_s.at[k - 1], recv_sem=l_r.at[k - 1],
                            device_id=nbr(-k), device_id_type=MESH,
                        ).wait_send()

        flows1v = [(p, c, nb, s) for p, c, nb, s in
                   (flows1 if stage >= 2 else [])]
        unicast_sends(flows1v, Z)

        out_ref[pl.ds(base, HALF), :] = summed[:HALF]
        out_ref[pl.ds(base + HALF, HALF), :] = summed[HALF:]

        for pos, chunk_at, nbr, (r_s, r_r, l_s, l_r) in flows1v:
            for k in range(1, Z):
                @pl.when(pos - k >= 0)
                def _(k=k, pos=pos, chunk_at=chunk_at, nbr=nbr,
                      r_s=r_s, r_r=r_r):
                    pltpu.make_async_remote_copy(
                        src_ref=chunk_at(pos - k), dst_ref=chunk_at(pos - k),
                        send_sem=r_s.at[k - 1], recv_sem=r_r.at[k - 1],
                        device_id=nbr(-k), device_id_type=MESH,
                    ).wait_recv()

                @pl.when(pos + k <= Z - 1)
                def _(k=k, pos=pos, chunk_at=chunk_at, nbr=nbr,
                      l_s=l_s, l_r=l_r):
                    pltpu.make_async_remote_copy(
                        src_ref=chunk_at(pos + k), dst_ref=chunk_at(pos + k),
                        send_sem=l_s.at[k - 1], recv_sem=l_r.at[k - 1],
                        device_id=nbr(k), device_id_type=MESH,
                    ).wait_recv()

        def conv_a_half(p):
            def go():
                val = a_ref[pl.ds(my_y, 1), pl.ds(p, 1), :, :]
                r0 = my_y * Z * BLK + p * BLK
                out_ref[pl.ds(r0, HALF), :] = (
                    val.reshape(HALF, D).astype(jnp.float32)
                )
            return go

        def conv_b_half(p):
            def go():
                val = b_ref[pl.ds(my_z, 1), pl.ds(p, 1), :, :]
                r0 = p * Z * BLK + my_z * BLK + HALF
                out_ref[pl.ds(r0, HALF), :] = (
                    val.reshape(HALF, D).astype(jnp.float32)
                )
            return go

        def conv_sub(f, p, zz):
            def go():
                if f == 0:
                    val = a_ref[pl.ds(p, 1), pl.ds(zz, 1), :, :]
                    r0 = p * Z * BLK + zz * BLK
                else:
                    val = b_ref[pl.ds(p, 1), pl.ds(zz, 1), :, :]
                    r0 = zz * Z * BLK + p * BLK + HALF
                out_ref[pl.ds(r0, HALF), :] = (
                    val.reshape(HALF, D).astype(jnp.float32)
                )
            return go

        def conv_plane_part(f, p):
            def go():
                for j in range(3):
                    conv_sub(f, p, my_x + j)()
            return go

        def guarded(cond, fn):
            def go():
                @pl.when(cond)
                def _():
                    fn()
            return go

        pending = []
        for k in (range(1, Z) if stage >= 2 else []):
            pending.append(guarded(my_z - k >= 0, conv_a_half(my_z - k)))
            pending.append(guarded(my_z + k <= Z - 1, conv_a_half(my_z + k)))
            pending.append(guarded(my_y - k >= 0, conv_b_half(my_y - k)))
            pending.append(guarded(my_y + k <= Y - 1, conv_b_half(my_y + k)))

        flows2v = [(p, c, fo, fi, nb, s) for p, c, fo, fi, nb, s in
                   (flows2 if stage >= 3 else [])]

        unicast_sends([(p, c, nb, s) for p, c, fo, fi, nb, s in flows2v], Y)

        for fn in pending:
            fn()
        pending = []

        for k in range(1, Y):
            for fi_, (pos, chunk_at, fout, fin, nbr,
                      (r_s, r_r, l_s, l_r)) in enumerate(flows2v):
                @pl.when(pos - k >= 0)
                def _(k=k, fi_=fi_, pos=pos, chunk_at=chunk_at, fout=fout,
                      nbr=nbr, r_s=r_s, r_r=r_r):
                    pltpu.make_async_remote_copy(
                        src_ref=chunk_at(pos - k), dst_ref=chunk_at(pos - k),
                        send_sem=r_s.at[k - 1], recv_sem=r_r.at[k - 1],
                        device_id=nbr(-k), device_id_type=MESH,
                    ).wait_recv()
                    pltpu.make_async_remote_copy(
                        src_ref=fout(pos - k), dst_ref=fout(pos - k),
                        send_sem=xf_send.at[fi_, 0, k - 1],
                        recv_sem=xf_recv.at[fi_, 0, k - 1],
                        device_id=partner, device_id_type=MESH,
                    ).start()

                @pl.when(pos + k <= Y - 1)
                def _(k=k, fi_=fi_, pos=pos, chunk_at=chunk_at, fout=fout,
                      nbr=nbr, l_s=l_s, l_r=l_r):
                    pltpu.make_async_remote_copy(
                        src_ref=chunk_at(pos + k), dst_ref=chunk_at(pos + k),
                        send_sem=l_s.at[k - 1], recv_sem=l_r.at[k - 1],
                        device_id=nbr(k), device_id_type=MESH,
                    ).wait_recv()
                    pltpu.make_async_remote_copy(
                        src_ref=fout(pos + k), dst_ref=fout(pos + k),
                        send_sem=xf_send.at[fi_, 1, k - 1],
                        recv_sem=xf_recv.at[fi_, 1, k - 1],
                        device_id=partner, device_id_type=MESH,
                    ).start()

            for fn in pending:
                fn()
            pending = []
            for fi_, (pos, chunk_at, fout, fin, nbr, _s) in enumerate(flows2v):
                pending.append(guarded(pos - k >= 0,
                                       conv_plane_part(fi_, pos - k)))
                pending.append(guarded(pos + k <= Y - 1,
                                       conv_plane_part(fi_, pos + k)))

        for k in range(1, Y):
            for fi_, (pos, chunk_at, fout, fin, nbr, _s) in enumerate(flows2v):
                @pl.when(pos - k >= 0)
                def _(k=k, fi_=fi_, pos=pos, fin=fin):
                    pltpu.make_async_remote_copy(
                        src_ref=fin(pos - k), dst_ref=fin(pos - k),
                        send_sem=xf_send.at[fi_, 0, k - 1],
                        recv_sem=xf_recv.at[fi_, 0, k - 1],
                        device_id=partner, device_id_type=MESH,
                    ).wait_recv()

                @pl.when(pos + k <= Y - 1)
                def _(k=k, fi_=fi_, pos=pos, fin=fin):
                    pltpu.make_async_remote_copy(
                        src_ref=fin(pos + k), dst_ref=fin(pos + k),
                        send_sem=xf_send.at[fi_, 1, k - 1],
                        recv_sem=xf_recv.at[fi_, 1, k - 1],
                        device_id=partner, device_id_type=MESH,
                    ).wait_recv()

        for fn in pending:
            fn()
        for k in (range(1, Y) if stage >= 3 else []):
            for fi_, (pos, chunk_at, fout, fin, nbr, _s) in enumerate(flows2v):
                guarded(pos - k >= 0, conv_sub(fi_, pos - k, fwd_in))()
                guarded(pos + k <= Y - 1, conv_sub(fi_, pos + k, fwd_in))()

        unicast_send_drains(flows1v, Z)
        unicast_send_drains(
            [(p, c, nb, s) for p, c, fo, fi, nb, s in flows2v], Y)
        for k in range(1, Y):
            for fi_, (pos, chunk_at, fout, fin, nbr, _s) in enumerate(flows2v):
                @pl.when(pos - k >= 0)
                def _(k=k, fi_=fi_, pos=pos, fout=fout):
                    pltpu.make_async_remote_copy(
                        src_ref=fout(pos - k), dst_ref=fout(pos - k),
                        send_sem=xf_send.at[fi_, 0, k - 1],
                        recv_sem=xf_recv.at[fi_, 0, k - 1],
                        device_id=partner, device_id_type=MESH,
                    ).wait_send()

                @pl.when(pos + k <= Y - 1)
                def _(k=k, fi_=fi_, pos=pos, fout=fout):
                    pltpu.make_async_remote_copy(
                        src_ref=fout(pos + k), dst_ref=fout(pos + k),
                        send_sem=xf_send.at[fi_, 1, k - 1],
                        recv_sem=xf_recv.at[fi_, 1, k - 1],
                        device_id=partner, device_id_type=MESH,
                    ).wait_send()

    ids2 = ids.reshape(T, 1)
    return pl.pallas_call(
        body,
        out_shape=jax.ShapeDtypeStruct((T, D), jnp.float32),
        in_specs=[
            pl.BlockSpec(memory_space=pltpu.SMEM),
            pl.BlockSpec(memory_space=pltpu.VMEM),
            pl.BlockSpec(memory_space=pl.ANY),
        ],
        out_specs=pl.BlockSpec(memory_space=pltpu.VMEM),
        scratch_shapes=[
            pltpu.VMEM((BLK, D), jnp.float32),
            pltpu.VMEM((BLK, D), jnp.bfloat16),
            pltpu.VMEM((BLK, D), jnp.bfloat16),
            pltpu.VMEM((Y, Z, HALF, D), jnp.bfloat16),
            pltpu.VMEM((Z, Y, HALF, D), jnp.bfloat16),
            pltpu.SemaphoreType.DMA,
            pltpu.SemaphoreType.DMA,
            pltpu.SemaphoreType.DMA,
            pltpu.SemaphoreType.DMA((8, Z - 1)),
            pltpu.SemaphoreType.DMA((8, Y - 1)),
            pltpu.SemaphoreType.DMA((2, 2, Y - 1)),
            pltpu.SemaphoreType.DMA((2, 2, Y - 1)),
        ],
        compiler_params=pltpu.CompilerParams(collective_id=0),
    )(ids, ids2, E)


# device time: 45166 ns/iter; 1.1969x vs baseline; 1.1969x over previous
import jax
import jax.numpy as jnp
from jax import lax
from jax.experimental import pallas as pl
from jax.experimental.pallas import tpu as pltpu

T = 2048
D = 1024
Y = 4
Z = 4
NB = Y * Z
BLK = T // NB
HALF = BLK // 2
MESH = pl.DeviceIdType.MESH


def kernel(ids, E, stage=3):
    V = E.shape[0]

    def body(ids_smem, ids_vmem, e_hbm, out_ref,
             own_ref, sendx_ref, recvx_ref, a_ref, b_ref,
             gather_sem, sendx_sem, recvx_sem, p1_sems, p2_sems):
        my_x = lax.axis_index("x")
        my_y = lax.axis_index("y")
        my_z = lax.axis_index("z")
        partner = (1 - my_x, my_y, my_z)
        offset = my_x * V
        b = my_y * Z + my_z
        base = b * BLK

        barrier = pltpu.get_barrier_semaphore()
        pl.semaphore_signal(barrier, inc=1, device_id=partner,
                            device_id_type=MESH)
        n_nbr = jnp.int32(1)
        for k in range(1, max(Y, Z)):
            for cond, tgt in [
                (my_z - k >= 0, (my_x, my_y, my_z - k)),
                (my_z + k <= Z - 1, (my_x, my_y, my_z + k)),
                (my_y - k >= 0, (my_x, my_y - k, my_z)),
                (my_y + k <= Y - 1, (my_x, my_y + k, my_z)),
            ]:
                @pl.when(cond)
                def _(tgt=tgt):
                    pl.semaphore_signal(barrier, inc=1, device_id=tgt,
                                        device_id_type=MESH)
                n_nbr = n_nbr + cond.astype(jnp.int32)

        UNROLL = 4

        def issue(i, carry):
            for j in range(UNROLL):
                t = i * UNROLL + j
                local = ids_smem[base + t] - offset
                c = jnp.clip(local, 0, V - 1)
                pltpu.make_async_copy(
                    e_hbm.at[pl.ds(c, 1), :],
                    own_ref.at[pl.ds(t, 1), :],
                    gather_sem,
                ).start()
            return carry

        lax.fori_loop(0, BLK // UNROLL, issue, 0)
        pltpu.make_async_copy(
            e_hbm.at[pl.ds(0, BLK), :], own_ref.at[:, :], gather_sem
        ).wait()

        ids_b = ids_vmem[pl.ds(base, BLK), :]
        in_range = (ids_b >= offset) & (ids_b < offset + V)
        masked = jnp.where(in_range, own_ref[:, :], 0.0)
        sendx_ref[:, :] = masked.astype(jnp.bfloat16)

        pl.semaphore_wait(barrier, n_nbr)

        xr = pltpu.make_async_remote_copy(
            src_ref=sendx_ref, dst_ref=recvx_ref,
            send_sem=sendx_sem, recv_sem=recvx_sem,
            device_id=partner, device_id_type=MESH,
        )
        xr.start()
        xr.wait()
        summed = masked + recvx_ref[:, :].astype(jnp.float32)
        summed_bf = summed.astype(jnp.bfloat16)
        a_ref[pl.ds(my_y, 1), pl.ds(my_z, 1), :, :] = (
            summed_bf[:HALF][None, None]
        )
        b_ref[pl.ds(my_z, 1), pl.ds(my_y, 1), :, :] = (
            summed_bf[HALF:][None, None]
        )

        z_nbr = lambda d: (my_x, my_y, my_z + d)
        y_nbr = lambda d: (my_x, my_y + d, my_z)
        a1_chunk = lambda p: a_ref.at[my_y, p]
        b1_chunk = lambda p: b_ref.at[my_z, p]
        a2_chunk = lambda p: a_ref.at[p]
        b2_chunk = lambda p: b_ref.at[p]
        sems1 = [tuple(p1_sems.at[i] for i in range(4 * f, 4 * f + 4))
                 for f in range(2)]
        sems2 = [tuple(p2_sems.at[i] for i in range(4 * f, 4 * f + 4))
                 for f in range(2)]
        flows1 = [(my_z, a1_chunk, z_nbr, sems1[0]),
                  (my_y, b1_chunk, y_nbr, sems1[1])]
        flows2 = [(my_y, a2_chunk, y_nbr, sems2[0]),
                  (my_z, b2_chunk, z_nbr, sems2[1])]


        for pos, chunk_at, nbr, (r_s, r_r, l_s, l_r) in (flows1 if stage >= 2 else []):
            for k in range(1, Z):
                @pl.when(pos + k <= Z - 1)
                def _(k=k, pos=pos, chunk_at=chunk_at, nbr=nbr,
                      r_s=r_s, r_r=r_r):
                    pltpu.make_async_remote_copy(
                        src_ref=chunk_at(pos), dst_ref=chunk_at(pos),
                        send_sem=r_s.at[k - 1], recv_sem=r_r.at[k - 1],
                        device_id=nbr(k), device_id_type=MESH,
                    ).start()

                @pl.when(pos - k >= 0)
                def _(k=k, pos=pos, chunk_at=chunk_at, nbr=nbr,
                      l_s=l_s, l_r=l_r):
                    pltpu.make_async_remote_copy(
                        src_ref=chunk_at(pos), dst_ref=chunk_at(pos),
                        send_sem=l_s.at[k - 1], recv_sem=l_r.at[k - 1],
                        device_id=nbr(-k), device_id_type=MESH,
                    ).start()

        out_ref[pl.ds(base, HALF), :] = summed[:HALF]
        out_ref[pl.ds(base + HALF, HALF), :] = summed[HALF:]

        for pos, chunk_at, nbr, (r_s, r_r, l_s, l_r) in (flows1 if stage >= 2 else []):
            for k in range(1, Z):
                @pl.when(pos - k >= 0)
                def _(k=k, pos=pos, chunk_at=chunk_at, nbr=nbr,
                      r_s=r_s, r_r=r_r):
                    pltpu.make_async_remote_copy(
                        src_ref=chunk_at(pos - k), dst_ref=chunk_at(pos - k),
                        send_sem=r_s.at[k - 1], recv_sem=r_r.at[k - 1],
                        device_id=nbr(-k), device_id_type=MESH,
                    ).wait_recv()

                @pl.when(pos + k <= Z - 1)
                def _(k=k, pos=pos, chunk_at=chunk_at, nbr=nbr,
                      l_s=l_s, l_r=l_r):
                    pltpu.make_async_remote_copy(
                        src_ref=chunk_at(pos + k), dst_ref=chunk_at(pos + k),
                        send_sem=l_s.at[k - 1], recv_sem=l_r.at[k - 1],
                        device_id=nbr(k), device_id_type=MESH,
                    ).wait_recv()

        def conv_a_half(p):
            def go():
                val = a_ref[pl.ds(my_y, 1), pl.ds(p, 1), :, :]
                r0 = my_y * Z * BLK + p * BLK
                out_ref[pl.ds(r0, HALF), :] = (
                    val.reshape(HALF, D).astype(jnp.float32)
                )
            return go

        def conv_b_half(p):
            def go():
                val = b_ref[pl.ds(my_z, 1), pl.ds(p, 1), :, :]
                r0 = p * Z * BLK + my_z * BLK + HALF
                out_ref[pl.ds(r0, HALF), :] = (
                    val.reshape(HALF, D).astype(jnp.float32)
                )
            return go

        def conv_a_strip(p):
            def go():
                for zz in range(Z):
                    val = a_ref[pl.ds(p, 1), pl.ds(zz, 1), :, :]
                    r0 = p * Z * BLK + zz * BLK
                    out_ref[pl.ds(r0, HALF), :] = (
                        val.reshape(HALF, D).astype(jnp.float32)
                    )
            return go

        def conv_b_strip(p):
            def go():
                for yy in range(Y):
                    val = b_ref[pl.ds(p, 1), pl.ds(yy, 1), :, :]
                    r0 = yy * Z * BLK + p * BLK + HALF
                    out_ref[pl.ds(r0, HALF), :] = (
                        val.reshape(HALF, D).astype(jnp.float32)
                    )
            return go

        def guarded(cond, fn):
            def go():
                @pl.when(cond)
                def _():
                    fn()
            return go

        pending = []
        for k in (range(1, Z) if stage >= 2 else []):
            pending.append(guarded(my_z - k >= 0, conv_a_half(my_z - k)))
            pending.append(guarded(my_z + k <= Z - 1, conv_a_half(my_z + k)))
            pending.append(guarded(my_y - k >= 0, conv_b_half(my_y - k)))
            pending.append(guarded(my_y + k <= Y - 1, conv_b_half(my_y + k)))

        csub = [lambda p, sub: a_ref.at[p, pl.ds(2 * sub, 2)],
                lambda p, sub: b_ref.at[p, pl.ds(2 * sub, 2)]]

        for fidx, (pos, chunk_at, nbr, (r_s, r_r, l_s, l_r)) in (
                enumerate(flows2) if stage >= 3 else []):
            for sub in range(2):
                @pl.when(pos < Y - 1)
                def _(fidx=fidx, sub=sub, pos=pos, nbr=nbr,
                      r_s=r_s, r_r=r_r):
                    pltpu.make_async_remote_copy(
                        src_ref=csub[fidx](pos, sub),
                        dst_ref=csub[fidx](pos, sub),
                        send_sem=r_s.at[0, sub], recv_sem=r_r.at[0, sub],
                        device_id=nbr(1), device_id_type=MESH,
                    ).start()

                @pl.when(pos > 0)
                def _(fidx=fidx, sub=sub, pos=pos, nbr=nbr,
                      l_s=l_s, l_r=l_r):
                    pltpu.make_async_remote_copy(
                        src_ref=csub[fidx](pos, sub),
                        dst_ref=csub[fidx](pos, sub),
                        send_sem=l_s.at[0, sub], recv_sem=l_r.at[0, sub],
                        device_id=nbr(-1), device_id_type=MESH,
                    ).start()

        for s in (range(Y - 1) if stage >= 3 else []):
            for fn in pending:
                fn()
            pending = []

            for sub in range(2):
                for fidx, (pos, chunk_at, nbr,
                           (r_s, r_r, l_s, l_r)) in enumerate(flows2):
                    @pl.when((pos > 0) & (pos - 1 - s >= 0))
                    def _(s=s, fidx=fidx, sub=sub, pos=pos, nbr=nbr,
                          r_s=r_s, r_r=r_r):
                        pltpu.make_async_remote_copy(
                            src_ref=csub[fidx](pos - 1 - s, sub),
                            dst_ref=csub[fidx](pos - 1 - s, sub),
                            send_sem=r_s.at[s, sub], recv_sem=r_r.at[s, sub],
                            device_id=nbr(-1), device_id_type=MESH,
                        ).wait_recv()

                    if s + 1 <= Y - 2:
                        @pl.when((pos < Y - 1) & (pos - 1 - s >= 0))
                        def _(s=s, fidx=fidx, sub=sub, pos=pos, nbr=nbr,
                              r_s=r_s, r_r=r_r):
                            pltpu.make_async_remote_copy(
                                src_ref=csub[fidx](pos - 1 - s, sub),
                                dst_ref=csub[fidx](pos - 1 - s, sub),
                                send_sem=r_s.at[s + 1, sub],
                                recv_sem=r_r.at[s + 1, sub],
                                device_id=nbr(1), device_id_type=MESH,
                            ).start()

                    @pl.when((pos < Y - 1) & (pos + 1 + s <= Y - 1))
                    def _(s=s, fidx=fidx, sub=sub, pos=pos, nbr=nbr,
                          l_s=l_s, l_r=l_r):
                        pltpu.make_async_remote_copy(
                            src_ref=csub[fidx](pos + 1 + s, sub),
                            dst_ref=csub[fidx](pos + 1 + s, sub),
                            send_sem=l_s.at[s, sub], recv_sem=l_r.at[s, sub],
                            device_id=nbr(1), device_id_type=MESH,
                        ).wait_recv()

                    if s + 1 <= Y - 2:
                        @pl.when((pos > 0) & (pos + 1 + s <= Y - 1))
                        def _(s=s, fidx=fidx, sub=sub, pos=pos, nbr=nbr,
                              l_s=l_s, l_r=l_r):
                            pltpu.make_async_remote_copy(
                                src_ref=csub[fidx](pos + 1 + s, sub),
                                dst_ref=csub[fidx](pos + 1 + s, sub),
                                send_sem=l_s.at[s + 1, sub],
                                recv_sem=l_r.at[s + 1, sub],
                                device_id=nbr(-1), device_id_type=MESH,
                            ).start()

            pending.append(guarded((my_y > 0) & (my_y - 1 - s >= 0),
                                   conv_a_strip(my_y - 1 - s)))
            pending.append(guarded((my_y < Y - 1) & (my_y + 1 + s <= Y - 1),
                                   conv_a_strip(my_y + 1 + s)))
            pending.append(guarded((my_z > 0) & (my_z - 1 - s >= 0),
                                   conv_b_strip(my_z - 1 - s)))
            pending.append(guarded((my_z < Z - 1) & (my_z + 1 + s <= Z - 1),
                                   conv_b_strip(my_z + 1 + s)))

        for fn in pending:
            fn()

        for pos, chunk_at, nbr, (r_s, r_r, l_s, l_r) in (flows1 if stage >= 2 else []):
            for k in range(1, Z):
                @pl.when(pos + k <= Z - 1)
                def _(k=k, pos=pos, chunk_at=chunk_at, nbr=nbr,
                      r_s=r_s, r_r=r_r):
                    pltpu.make_async_remote_copy(
                        src_ref=chunk_at(pos), dst_ref=chunk_at(pos),
                        send_sem=r_s.at[k - 1], recv_sem=r_r.at[k - 1],
                        device_id=nbr(k), device_id_type=MESH,
                    ).wait_send()

                @pl.when(pos - k >= 0)
                def _(k=k, pos=pos, chunk_at=chunk_at, nbr=nbr,
                      l_s=l_s, l_r=l_r):
                    pltpu.make_async_remote_copy(
                        src_ref=chunk_at(pos), dst_ref=chunk_at(pos),
                        send_sem=l_s.at[k - 1], recv_sem=l_r.at[k - 1],
                        device_id=nbr(-k), device_id_type=MESH,
                    ).wait_send()

        for s in (range(Y - 1) if stage >= 3 else []):
            for sub in range(2):
                for fidx, (pos, chunk_at, nbr,
                           (r_s, r_r, l_s, l_r)) in enumerate(flows2):
                    @pl.when((pos < Y - 1) & (pos - s >= 0))
                    def _(s=s, fidx=fidx, sub=sub, pos=pos, nbr=nbr,
                          r_s=r_s, r_r=r_r):
                        pltpu.make_async_remote_copy(
                            src_ref=csub[fidx](pos - s, sub),
                            dst_ref=csub[fidx](pos - s, sub),
                            send_sem=r_s.at[s, sub], recv_sem=r_r.at[s, sub],
                            device_id=nbr(1), device_id_type=MESH,
                        ).wait_send()

                    @pl.when((pos > 0) & (pos + s <= Y - 1))
                    def _(s=s, fidx=fidx, sub=sub, pos=pos, nbr=nbr,
                          l_s=l_s, l_r=l_r):
                        pltpu.make_async_remote_copy(
                            src_ref=csub[fidx](pos + s, sub),
                            dst_ref=csub[fidx](pos + s, sub),
                            send_sem=l_s.at[s, sub], recv_sem=l_r.at[s, sub],
                            device_id=nbr(-1), device_id_type=MESH,
                        ).wait_send()

    ids2 = ids.reshape(T, 1)
    return pl.pallas_call(
        body,
        out_shape=jax.ShapeDtypeStruct((T, D), jnp.float32),
        in_specs=[
            pl.BlockSpec(memory_space=pltpu.SMEM),
            pl.BlockSpec(memory_space=pltpu.VMEM),
            pl.BlockSpec(memory_space=pl.ANY),
        ],
        out_specs=pl.BlockSpec(memory_space=pltpu.VMEM),
        scratch_shapes=[
            pltpu.VMEM((BLK, D), jnp.float32),
            pltpu.VMEM((BLK, D), jnp.bfloat16),
            pltpu.VMEM((BLK, D), jnp.bfloat16),
            pltpu.VMEM((Y, Z, HALF, D), jnp.bfloat16),
            pltpu.VMEM((Z, Y, HALF, D), jnp.bfloat16),
            pltpu.SemaphoreType.DMA,
            pltpu.SemaphoreType.DMA,
            pltpu.SemaphoreType.DMA,
            pltpu.SemaphoreType.DMA((8, Z - 1)),
            pltpu.SemaphoreType.DMA((8, Y - 1, 2)),
        ],
        compiler_params=pltpu.CompilerParams(collective_id=0),
    )(ids, ids2, E)
